# baseline (device time: 137395 ns/iter reference)
import jax
import jax.numpy as jnp
from jax import lax
from jax.experimental import pallas as pl
from jax.experimental.pallas import tpu as pltpu

T = 4096
D = 1024
CH = 512
N_MAX = T // CH
UNROLL = 8
ROW = (8, 128)


def _body(cnt_ref, order_ref, x_ref, out_ref, xs_ref, send_sems, recv_sems):
    my_x = lax.axis_index("x")
    my_y = lax.axis_index("y")
    my_z = lax.axis_index("z")
    peer = (my_x, my_y, 1 - my_z)
    cnt0 = cnt_ref[0]

    is0 = my_z == 0
    send_count = jnp.where(is0, T - cnt0, cnt0)
    keep_count = T - send_count
    n_send = (send_count + CH - 1) // CH
    n_keep = (keep_count + CH - 1) // CH
    dst_shift = jnp.where(is0, -cnt0, T - cnt0)

    def gather_rows(dst_ref, start):
        def grp(g, _):
            base = start + g * UNROLL
            for u in range(UNROLL):
                j = base + u
                dst_ref[pl.ds(j, 1)] = x_ref[pl.ds(order_ref[j], 1)]
            return 0

        lax.fori_loop(0, CH // UNROLL, grp, 0)

    for i in range(N_MAX):
        src_start = jnp.where(
            is0,
            jnp.maximum(T - (i + 1) * CH, cnt0),
            jnp.minimum(i * CH, cnt0 - CH),
        )
        dst_start = src_start + dst_shift

        @pl.when(i < n_send)
        def _(i=i, src_start=src_start, dst_start=dst_start):
            gather_rows(xs_ref, src_start)
            pltpu.make_async_remote_copy(
                src_ref=xs_ref.at[pl.ds(src_start, CH)],
                dst_ref=out_ref.at[pl.ds(dst_start, CH)],
                send_sem=send_sems.at[i],
                recv_sem=recv_sems.at[i],
                device_id=peer,
                device_id_type=pl.DeviceIdType.MESH,
            ).start()

    for i in range(N_MAX):
        start = jnp.where(
            is0,
            jnp.minimum(i * CH, cnt0 - CH),
            jnp.maximum(T - (i + 1) * CH, cnt0),
        )

        @pl.when(i < n_keep)
        def _(start=start):
            gather_rows(out_ref, start)

    for i in range(N_MAX):

        @pl.when(i < n_send)
        def _(i=i):
            pltpu.make_async_remote_copy(
                src_ref=xs_ref.at[pl.ds(0, CH)],
                dst_ref=out_ref.at[pl.ds(0, CH)],
                send_sem=send_sems.at[i],
                recv_sem=recv_sems.at[i],
                device_id=peer,
                device_id_type=pl.DeviceIdType.MESH,
            ).wait_recv()

    for i in range(N_MAX):

        @pl.when(i < n_send)
        def _(i=i):
            pltpu.make_async_remote_copy(
                src_ref=xs_ref.at[pl.ds(0, CH)],
                dst_ref=out_ref.at[pl.ds(0, CH)],
                send_sem=send_sems.at[i],
                recv_sem=recv_sems.at[i],
                device_id=peer,
                device_id_type=pl.DeviceIdType.MESH,
            ).wait_send()


def kernel(x, dest):
    order = jnp.argsort(dest, stable=True).astype(jnp.int32)
    cnt0 = jnp.sum(dest == 0).astype(jnp.int32).reshape((1,))

    out = pl.pallas_call(
        _body,
        out_shape=jax.ShapeDtypeStruct((T, *ROW), jnp.float32),
        in_specs=[
            pl.BlockSpec(memory_space=pltpu.SMEM),
            pl.BlockSpec(memory_space=pltpu.SMEM),
            pl.BlockSpec(memory_space=pltpu.VMEM),
        ],
        out_specs=pl.BlockSpec(memory_space=pltpu.VMEM),
        scratch_shapes=[
            pltpu.VMEM((T, *ROW), jnp.float32),
            pltpu.SemaphoreType.DMA((N_MAX,)),
            pltpu.SemaphoreType.DMA((N_MAX,)),
        ],
    )(cnt0, order, x.reshape(T, *ROW))
    return out.reshape(T, D)


# device time: 133127 ns/iter; 1.0321x vs baseline; 1.0321x over previous
import jax
import jax.numpy as jnp
from jax import lax
from jax.experimental import pallas as pl
from jax.experimental.pallas import tpu as pltpu

T = 4096
D = 1024
CH = 512
N_MAX = T // CH
UNROLL = 8
ROW = (8, 128)


def _body(cnt_ref, order_ref, x_ref, out_ref, xs_ref, send_sems, recv_sems):
    my_x = lax.axis_index("x")
    my_y = lax.axis_index("y")
    my_z = lax.axis_index("z")
    peer = (my_x, my_y, 1 - my_z)
    cnt0 = cnt_ref[0]

    barrier_sem = pltpu.get_barrier_semaphore()
    pl.semaphore_signal(
        barrier_sem, inc=1, device_id=peer,
        device_id_type=pl.DeviceIdType.MESH,
    )
    pl.semaphore_wait(barrier_sem, 1)

    is0 = my_z == 0
    send_count = jnp.where(is0, T - cnt0, cnt0)
    keep_count = T - send_count
    n_send = (send_count + CH - 1) // CH
    n_keep = (keep_count + CH - 1) // CH
    dst_shift = jnp.where(is0, -cnt0, T - cnt0)

    def gather_rows(dst_ref, start):
        def grp(g, _):
            base = start + g * UNROLL
            for u in range(UNROLL):
                j = base + u
                dst_ref[pl.ds(j, 1)] = x_ref[pl.ds(order_ref[j], 1)]
            return 0

        lax.fori_loop(0, CH // UNROLL, grp, 0)

    for i in range(N_MAX):
        src_start = jnp.where(
            is0,
            jnp.maximum(T - (i + 1) * CH, cnt0),
            jnp.minimum(i * CH, cnt0 - CH),
        )
        dst_start = src_start + dst_shift

        @pl.when(i < n_send)
        def _(i=i, src_start=src_start, dst_start=dst_start):
            gather_rows(xs_ref, src_start)
            pltpu.make_async_remote_copy(
                src_ref=xs_ref.at[pl.ds(src_start, CH)],
                dst_ref=out_ref.at[pl.ds(dst_start, CH)],
                send_sem=send_sems.at[i],
                recv_sem=recv_sems.at[i],
                device_id=peer,
                device_id_type=pl.DeviceIdType.MESH,
            ).start()

    for i in range(N_MAX):
        start = jnp.where(
            is0,
            jnp.minimum(i * CH, cnt0 - CH),
            jnp.maximum(T - (i + 1) * CH, cnt0),
        )

        @pl.when(i < n_keep)
        def _(start=start):
            gather_rows(out_ref, start)

    for i in range(N_MAX):

        @pl.when(i < n_send)
        def _(i=i):
            pltpu.make_async_remote_copy(
                src_ref=xs_ref.at[pl.ds(0, CH)],
                dst_ref=out_ref.at[pl.ds(0, CH)],
                send_sem=send_sems.at[i],
                recv_sem=recv_sems.at[i],
                device_id=peer,
                device_id_type=pl.DeviceIdType.MESH,
            ).wait_recv()

    for i in range(N_MAX):

        @pl.when(i < n_send)
        def _(i=i):
            pltpu.make_async_remote_copy(
                src_ref=xs_ref.at[pl.ds(0, CH)],
                dst_ref=out_ref.at[pl.ds(0, CH)],
                send_sem=send_sems.at[i],
                recv_sem=recv_sems.at[i],
                device_id=peer,
                device_id_type=pl.DeviceIdType.MESH,
            ).wait_send()


def kernel(x, dest):
    order = jnp.argsort(dest, stable=True).astype(jnp.int32)
    cnt0 = jnp.sum(dest == 0).astype(jnp.int32).reshape((1,))

    out = pl.pallas_call(
        _body,
        out_shape=jax.ShapeDtypeStruct((T, *ROW), jnp.float32),
        in_specs=[
            pl.BlockSpec(memory_space=pltpu.SMEM),
            pl.BlockSpec(memory_space=pltpu.SMEM),
            pl.BlockSpec(memory_space=pltpu.VMEM),
        ],
        out_specs=pl.BlockSpec(memory_space=pltpu.VMEM),
        scratch_shapes=[
            pltpu.VMEM((T, *ROW), jnp.float32),
            pltpu.SemaphoreType.DMA((N_MAX,)),
            pltpu.SemaphoreType.DMA((N_MAX,)),
        ],
        compiler_params=pltpu.CompilerParams(collective_id=0),
    )(cnt0, order, x.reshape(T, *ROW))
    return out.reshape(T, D)


# device time: 120190 ns/iter; 1.1431x vs baseline; 1.1076x over previous
import jax
import jax.numpy as jnp
from jax import lax
from jax.experimental import pallas as pl
from jax.experimental.pallas import tpu as pltpu

T = 4096
D = 1024
CH = 512
N_MAX = T // CH
UNROLL = 8
ROW = (8, 128)


def _body(cnt_ref, order_ref, x_ref, out_ref, xs_ref, send_sems, recv_sems):
    my_x = lax.axis_index("x")
    my_y = lax.axis_index("y")
    my_z = lax.axis_index("z")
    peer = (my_x, my_y, 1 - my_z)
    cnt0 = cnt_ref[0]

    barrier_sem = pltpu.get_barrier_semaphore()
    pl.semaphore_signal(
        barrier_sem, inc=1, device_id=peer,
        device_id_type=pl.DeviceIdType.MESH,
    )
    pl.semaphore_wait(barrier_sem, 1)

    is0 = my_z == 0
    send_count = jnp.where(is0, T - cnt0, cnt0)
    keep_count = T - send_count
    n_send = (send_count + CH - 1) // CH
    n_keep = (keep_count + CH - 1) // CH
    dst_shift = jnp.where(is0, -cnt0, T - cnt0)

    def gather_rows(dst_ref, start):
        def grp(g, _):
            base = start + g * UNROLL
            for u in range(UNROLL):
                j = base + u
                dst_ref[pl.ds(j, 1)] = x_ref[
                    pl.ds(order_ref[j], 1), :
                ].reshape(1, *ROW)
            return 0

        lax.fori_loop(0, CH // UNROLL, grp, 0)

    for i in range(N_MAX):
        src_start = jnp.where(
            is0,
            jnp.maximum(T - (i + 1) * CH, cnt0),
            jnp.minimum(i * CH, cnt0 - CH),
        )
        dst_start = src_start + dst_shift

        @pl.when(i < n_send)
        def _(i=i, src_start=src_start, dst_start=dst_start):
            gather_rows(xs_ref, src_start)
            pltpu.make_async_remote_copy(
                src_ref=xs_ref.at[pl.ds(src_start, CH)],
                dst_ref=out_ref.at[pl.ds(dst_start, CH)],
                send_sem=send_sems.at[i],
                recv_sem=recv_sems.at[i],
                device_id=peer,
                device_id_type=pl.DeviceIdType.MESH,
            ).start()

    for i in range(N_MAX):
        start = jnp.where(
            is0,
            jnp.minimum(i * CH, cnt0 - CH),
            jnp.maximum(T - (i + 1) * CH, cnt0),
        )

        @pl.when(i < n_keep)
        def _(start=start):
            gather_rows(out_ref, start)

    for i in range(N_MAX):

        @pl.when(i < n_send)
        def _(i=i):
            pltpu.make_async_remote_copy(
                src_ref=xs_ref.at[pl.ds(0, CH)],
                dst_ref=out_ref.at[pl.ds(0, CH)],
                send_sem=send_sems.at[i],
                recv_sem=recv_sems.at[i],
                device_id=peer,
                device_id_type=pl.DeviceIdType.MESH,
            ).wait_recv()

    for i in range(N_MAX):

        @pl.when(i < n_send)
        def _(i=i):
            pltpu.make_async_remote_copy(
                src_ref=xs_ref.at[pl.ds(0, CH)],
                dst_ref=out_ref.at[pl.ds(0, CH)],
                send_sem=send_sems.at[i],
                recv_sem=recv_sems.at[i],
                device_id=peer,
                device_id_type=pl.DeviceIdType.MESH,
            ).wait_send()


def kernel(x, dest):
    order = jnp.argsort(dest, stable=True).astype(jnp.int32)
    cnt0 = jnp.sum(dest == 0).astype(jnp.int32).reshape((1,))

    out = pl.pallas_call(
        _body,
        out_shape=jax.ShapeDtypeStruct((T, *ROW), jnp.float32),
        in_specs=[
            pl.BlockSpec(memory_space=pltpu.SMEM),
            pl.BlockSpec(memory_space=pltpu.SMEM),
            pl.BlockSpec(memory_space=pltpu.VMEM),
        ],
        out_specs=pl.BlockSpec(memory_space=pltpu.VMEM),
        scratch_shapes=[
            pltpu.VMEM((T, *ROW), jnp.float32),
            pltpu.SemaphoreType.DMA((N_MAX,)),
            pltpu.SemaphoreType.DMA((N_MAX,)),
        ],
        compiler_params=pltpu.CompilerParams(collective_id=0),
    )(cnt0, order, x)
    return out.reshape(T, D)
